# baseline (speedup 1.0000x reference)
import functools

import jax
import jax.numpy as jnp
from jax import lax
from jax.experimental import pallas as pl
from jax.experimental.pallas import tpu as pltpu
from jax.experimental.pallas import tpu_sc as plsc

EPS = 1e-05
LANE = 128


def _log0(y, sqrt_c):
    diff = y / (1.0 + EPS)
    dn = jnp.sqrt(jnp.sum(diff * diff, axis=-1, keepdims=True))
    dn = jnp.maximum(dn, EPS)
    arg = jnp.minimum(sqrt_c * dn, 1.0 - 1e-06)
    atanh = 0.5 * jnp.log((1.0 + arg) / (1.0 - arg))
    return (atanh / (sqrt_c * dn)) * diff


def _exp0(v, sqrt_c):
    vn = jnp.sqrt(jnp.sum(v * v, axis=-1, keepdims=True))
    vn = jnp.maximum(vn, EPS)
    factor = jnp.tanh(sqrt_c * vn) / (sqrt_c * vn)
    return (factor * v) / (1.0 + EPS)


def _mobius_add(x, y, c):
    x2 = jnp.sum(x * x, axis=-1, keepdims=True)
    y2 = jnp.sum(y * y, axis=-1, keepdims=True)
    xy = jnp.sum(x * y, axis=-1, keepdims=True)
    num = (1.0 + 2.0 * c * xy + c * y2) * x + (1.0 - c * x2) * y
    den = 1.0 + 2.0 * c * xy + c * x2 * y2
    return num / (den + EPS)


def _two_branch(t, wnt, bn, wst, bs, sqrt_c):
    y = _exp0(jnp.dot(t, wnt, preferred_element_type=jnp.float32) + bn, sqrt_c)
    xs = _exp0(jnp.dot(t, wst, preferred_element_type=jnp.float32) + bs, sqrt_c)
    return y, xs


def _pre_body(c_ref, x_ref, wpt, bp, wnt, bn, wst, bs, y_ref, xs_ref):
    cc = c_ref[0, 0]
    sq = jnp.sqrt(cc)
    x = x_ref[...]
    nrm = jnp.sqrt(jnp.sum(x * x, axis=-1, keepdims=True))
    x = x / (nrm + 1e-08)
    h = jnp.dot(_log0(x, sq), wpt[...], preferred_element_type=jnp.float32) + bp[...]
    x0 = _exp0(h, sq)
    t = _log0(x0, sq)
    y_ref[...], xs_ref[...] = _two_branch(t, wnt[...], bn[...], wst[...], bs[...], sq)


def _layer_update(c_ref, xs_ref, y_ref, p_ref, d_ref, nc):
    cc = c_ref[0, 0]
    sq = jnp.sqrt(cc)
    y = y_ref[...]
    s = y
    for k in range(nc):
        s = s + p_ref[k]
    deg = d_ref[0, :, 0:1]
    for k in range(1, nc):
        deg = deg + d_ref[k, :, 0:1]
    deg = jnp.maximum(deg + 1.0, 1.0)
    agg = s / deg
    xn = _mobius_add(xs_ref[...], agg, cc)
    xt = jnp.tanh(_log0(xn, sq))
    return _exp0(xt, sq), sq


def _mid_body(c_ref, xs_ref, y_ref, p_ref, d_ref, wnt, bn, wst, bs, y1_ref, xs1_ref, *, nc):
    x1, sq = _layer_update(c_ref, xs_ref, y_ref, p_ref, d_ref, nc)
    t = _log0(x1, sq)
    y1_ref[...], xs1_ref[...] = _two_branch(t, wnt[...], bn[...], wst[...], bs[...], sq)


def _post_body(c_ref, xs_ref, y_ref, p_ref, d_ref, w1t, b1, w2t, b2, out_ref, *, nc):
    x2, sq = _layer_update(c_ref, xs_ref, y_ref, p_ref, d_ref, nc)
    t = _log0(x2, sq)
    h = jnp.tanh(jnp.dot(t, w1t[...], preferred_element_type=jnp.float32) + b1[...])
    out_ref[...] = jnp.dot(h, w2t[...], preferred_element_type=jnp.float32) + b2[...]


def _row_spec(blk):
    return pl.BlockSpec((blk, LANE), lambda i: (i, 0))


def _w_spec():
    return pl.BlockSpec((LANE, LANE), lambda i: (0, 0))


def _b_spec():
    return pl.BlockSpec((LANE,), lambda i: (0,))


def _smem_spec():
    return pl.BlockSpec(memory_space=pltpu.SMEM)


def _p_spec(nc, blk):
    return pl.BlockSpec((nc, blk, LANE), lambda i: (0, i, 0))


def _d_spec(nc, blk):
    return pl.BlockSpec((nc, blk, LANE), lambda i: (0, i, 0))


def _make_seg_sum(npad, nch0, nch1, nc, ns):
    rows_pt = npad // ns
    mesh = plsc.VectorSubcoreMesh(core_axis_name="c", subcore_axis_name="s")

    out_type = [jax.ShapeDtypeStruct((nc, npad, LANE), jnp.float32)]
    scratch = [
        pltpu.VMEM_SHARED((npad, LANE), jnp.float32),
        pltpu.VMEM((LANE,), jnp.int32),
        pltpu.VMEM((LANE,), jnp.int32),
        pltpu.VMEM((LANE,), jnp.int32),
        pltpu.VMEM((LANE,), jnp.int32),
        pltpu.VMEM((LANE, LANE // 2), jnp.int32),
        pltpu.VMEM((LANE, LANE // 2), jnp.int32),
        pltpu.VMEM((LANE, LANE), jnp.float32),
        pltpu.SemaphoreType.DMA,
        pltpu.SemaphoreType.DMA,
    ]

    @functools.partial(
        pl.kernel, out_type=out_type, mesh=mesh, scratch_types=scratch,
        compiler_params=pltpu.CompilerParams(use_tc_tiling_on_sc=False))
    def seg(ypk_hbm, src0_hbm, dst0_hbm, src1_hbm, dst1_hbm, z_hbm, agg_out,
            agg_sh, si0, si1, di0, di1, r0, r1, fbuf, sem0, sem1):
        cid = lax.axis_index("c")
        sid = lax.axis_index("s")
        base = sid * rows_pt

        pltpu.sync_copy(z_hbm, agg_sh.at[pl.ds(base, rows_pt)])
        plsc.subcore_barrier()

        si = (si0, si1)
        di = (di0, di1)
        rows = (r0, r1)
        sems = (sem0, sem1)
        hi_mask = jnp.int32(-65536)

        def widen(rb):
            def conv_row(i, carry):
                for k in range(4):
                    w = rb[i, pl.ds(k * 16, 16)]
                    fbuf[i, pl.ds(32 * k, 16)] = lax.bitcast_convert_type(
                        w << 16, jnp.float32)
                    fbuf[i, pl.ds(32 * k + 16, 16)] = lax.bitcast_convert_type(
                        w & hi_mask, jnp.float32)
                return carry
            lax.fori_loop(0, LANE, conv_row, 0)

        def edge_loop(src_hbm, dst_hbm, n_chunks):
            pltpu.sync_copy(src_hbm.at[sid, 0], si0)
            pltpu.sync_copy(dst_hbm.at[sid, 0], di0)
            pltpu.async_copy(ypk_hbm.at[si0], r0, sem0)

            def loop_body(i2, carry):
                for b in (0, 1):
                    j = i2 * 2 + b
                    nb = 1 - b

                    @pl.when(j + 1 < n_chunks)
                    def _():
                        pltpu.sync_copy(src_hbm.at[sid, j + 1], si[nb])
                        pltpu.sync_copy(dst_hbm.at[sid, j + 1], di[nb])
                        pltpu.async_copy(ypk_hbm.at[si[nb]], rows[nb],
                                         sems[nb])

                    pltpu.make_async_copy(ypk_hbm.at[si[b]], rows[b],
                                          sems[b]).wait()
                    widen(rows[b])
                    pltpu.sync_copy(fbuf, agg_sh.at[di[b]], add=True)
                return carry

            lax.fori_loop(0, n_chunks // 2, loop_body, 0)

        @pl.when(cid == 0)
        def _():
            edge_loop(src0_hbm, dst0_hbm, nch0)

        @pl.when(cid == 1)
        def _():
            edge_loop(src1_hbm, dst1_hbm, nch1)

        plsc.subcore_barrier()

        pltpu.sync_copy(agg_sh.at[pl.ds(base, rows_pt)],
                        agg_out.at[cid, pl.ds(base, rows_pt)])

    return seg


def _make_deg(npad, n_chunks, nc, ns):
    rows_pt = npad // ns
    mesh = plsc.VectorSubcoreMesh(core_axis_name="c", subcore_axis_name="s")

    @functools.partial(
        pl.kernel,
        out_type=[jax.ShapeDtypeStruct((nc, npad, LANE), jnp.float32)],
        mesh=mesh,
        scratch_types=[
            pltpu.VMEM_SHARED((npad, LANE), jnp.float32),
            pltpu.VMEM((LANE,), jnp.int32),
            pltpu.VMEM((LANE,), jnp.int32),
            pltpu.VMEM((LANE, LANE), jnp.float32),
            pltpu.SemaphoreType.DMA,
            pltpu.SemaphoreType.DMA,
        ])
    def deg(dst_hbm, z_hbm, ones_hbm, deg_out,
            deg_sh, di0, di1, ones_v, sem0, sem1):
        cid = lax.axis_index("c")
        sid = lax.axis_index("s")
        wid = sid * nc + cid
        base = sid * rows_pt

        pltpu.sync_copy(z_hbm, deg_sh.at[pl.ds(base, rows_pt)])
        pltpu.sync_copy(ones_hbm, ones_v)
        plsc.subcore_barrier()

        di = (di0, di1)
        sems = (sem0, sem1)
        pltpu.async_copy(dst_hbm.at[wid, 0], di0, sem0)

        def loop_body(i2, carry):
            for b in (0, 1):
                j = i2 * 2 + b
                nb = 1 - b

                @pl.when(j + 1 < n_chunks)
                def _():
                    pltpu.async_copy(dst_hbm.at[wid, j + 1], di[nb], sems[nb])

                pltpu.make_async_copy(dst_hbm.at[wid, j], di[b], sems[b]).wait()
                pltpu.sync_copy(ones_v, deg_sh.at[di[b]], add=True)
            return carry

        lax.fori_loop(0, n_chunks // 2, loop_body, 0)
        plsc.subcore_barrier()
        pltpu.sync_copy(deg_sh.at[pl.ds(base, rows_pt)],
                        deg_out.at[cid, pl.ds(base, rows_pt)])

    return deg


def _pack_bf16_pairs(y, npad):
    y_pre = y.reshape(npad, 4, 2, 16).transpose(0, 1, 3, 2).reshape(npad, LANE)
    ybf = y_pre.astype(jnp.bfloat16)
    return jax.lax.bitcast_convert_type(
        ybf.reshape(npad, LANE // 2, 2), jnp.int32)


def kernel(x, c, Wp, bp, Ws0, bs0, Wn0, bn0, Ws1, bs1, Wn1, bn1, W1, b1,
           W2, b2, edge_index):
    N, D = x.shape
    H = Wp.shape[0]
    Cout = W2.shape[0]
    E = edge_index.shape[1]

    info = plsc.get_sparse_core_info()
    nc, ns = info.num_cores, info.num_subcores
    nw = nc * ns

    rows_pt = -(-(N + 1) // (ns * 8)) * 8
    npad = ns * rows_pt
    idt = edge_index.dtype

    n_chunks = max(2, -(-E // (nw * 2 * LANE)) * 2)
    pad_e = nw * n_chunks * LANE - E
    dstp = jnp.concatenate(
        [edge_index[1], jnp.full((pad_e,), N, idt)]).astype(jnp.int32).reshape(
            nw, n_chunks, LANE)

    frac0 = 0.55
    ch_tot = -(-E // LANE)
    nch0 = max(2, -(-int(ch_tot * frac0) // (ns * 2)) * 2)
    cap0 = ns * nch0 * LANE
    rem = max(0, E - cap0)
    nch1 = max(2, -(-rem // (LANE * ns * 2)) * 2)
    cap1 = ns * nch1 * LANE
    pad_s = cap0 + cap1 - E
    src_all = jnp.concatenate(
        [edge_index[0], jnp.full((pad_s,), N, idt)]).astype(jnp.int32)
    dst_all = jnp.concatenate(
        [edge_index[1], jnp.full((pad_s,), N, idt)]).astype(jnp.int32)
    src0 = src_all[:cap0].reshape(ns, nch0, LANE)
    dst0 = dst_all[:cap0].reshape(ns, nch0, LANE)
    src1 = src_all[cap0:].reshape(ns, nch1, LANE)
    dst1 = dst_all[cap0:].reshape(ns, nch1, LANE)

    xp = jnp.zeros((npad, D), jnp.float32).at[:N].set(x)
    c2 = jnp.reshape(c, (1, 1)).astype(jnp.float32)
    z128 = jnp.zeros((rows_pt, LANE), jnp.float32)
    o128 = jnp.ones((LANE, LANE), jnp.float32)

    w2pad = jnp.zeros((LANE, H), jnp.float32).at[:Cout].set(W2)
    b2pad = jnp.zeros((LANE,), jnp.float32).at[:Cout].set(b2)

    blk = 2 * rows_pt
    grid = (npad // blk,)

    f32 = jnp.float32
    rows2 = jax.ShapeDtypeStruct((npad, LANE), f32)

    y0, xs0 = pl.pallas_call(
        _pre_body,
        grid=grid,
        in_specs=[_smem_spec(), _row_spec(blk), _w_spec(), _b_spec(),
                  _w_spec(), _b_spec(), _w_spec(), _b_spec()],
        out_specs=[_row_spec(blk), _row_spec(blk)],
        out_shape=[rows2, rows2],
    )(c2, xp, Wp.T, bp, Wn0.T, bn0, Ws0.T, bs0)

    degk = _make_deg(npad, n_chunks, nc, ns)
    d0 = degk(dstp, z128, o128)
    if isinstance(d0, (list, tuple)):
        d0 = d0[0]

    seg0 = _make_seg_sum(npad, nch0, nch1, nc, ns)
    p0 = seg0(_pack_bf16_pairs(y0, npad), src0, dst0, src1, dst1, z128)
    if isinstance(p0, (list, tuple)):
        p0 = p0[0]

    y1, xs1 = pl.pallas_call(
        functools.partial(_mid_body, nc=nc),
        grid=grid,
        in_specs=[_smem_spec(), _row_spec(blk), _row_spec(blk),
                  _p_spec(nc, blk), _d_spec(nc, blk),
                  _w_spec(), _b_spec(), _w_spec(), _b_spec()],
        out_specs=[_row_spec(blk), _row_spec(blk)],
        out_shape=[rows2, rows2],
    )(c2, xs0, y0, p0, d0, Wn1.T, bn1, Ws1.T, bs1)

    seg1 = _make_seg_sum(npad, nch0, nch1, nc, ns)
    p1 = seg1(_pack_bf16_pairs(y1, npad), src0, dst0, src1, dst1, z128)
    if isinstance(p1, (list, tuple)):
        p1 = p1[0]

    out = pl.pallas_call(
        functools.partial(_post_body, nc=nc),
        grid=grid,
        in_specs=[_smem_spec(), _row_spec(blk), _row_spec(blk),
                  _p_spec(nc, blk), _d_spec(nc, blk),
                  _w_spec(), _b_spec(), _w_spec(), _b_spec()],
        out_specs=[_row_spec(blk)],
        out_shape=[rows2],
    )(c2, xs1, y1, p1, d0, W1.T, b1, w2pad.T, b2pad)
    if isinstance(out, (list, tuple)):
        out = out[0]

    return out[:N, :Cout]


# speedup vs baseline: 1.4186x; 1.4186x over previous
import functools

import jax
import jax.numpy as jnp
from jax import lax
from jax.experimental import pallas as pl
from jax.experimental.pallas import tpu as pltpu
from jax.experimental.pallas import tpu_sc as plsc

EPS = 1e-05
LANE = 128


def _log0(y, sqrt_c):
    diff = y / (1.0 + EPS)
    dn = jnp.sqrt(jnp.sum(diff * diff, axis=-1, keepdims=True))
    dn = jnp.maximum(dn, EPS)
    arg = jnp.minimum(sqrt_c * dn, 1.0 - 1e-06)
    atanh = 0.5 * jnp.log((1.0 + arg) / (1.0 - arg))
    return (atanh / (sqrt_c * dn)) * diff


def _exp0(v, sqrt_c):
    vn = jnp.sqrt(jnp.sum(v * v, axis=-1, keepdims=True))
    vn = jnp.maximum(vn, EPS)
    factor = jnp.tanh(sqrt_c * vn) / (sqrt_c * vn)
    return (factor * v) / (1.0 + EPS)


def _mobius_add(x, y, c):
    x2 = jnp.sum(x * x, axis=-1, keepdims=True)
    y2 = jnp.sum(y * y, axis=-1, keepdims=True)
    xy = jnp.sum(x * y, axis=-1, keepdims=True)
    num = (1.0 + 2.0 * c * xy + c * y2) * x + (1.0 - c * x2) * y
    den = 1.0 + 2.0 * c * xy + c * x2 * y2
    return num / (den + EPS)


def _two_branch(t, wnt, bn, wst, bs, sqrt_c):
    y = _exp0(jnp.dot(t, wnt, preferred_element_type=jnp.float32) + bn, sqrt_c)
    xs = _exp0(jnp.dot(t, wst, preferred_element_type=jnp.float32) + bs, sqrt_c)
    return y, xs


def _pre_body(c_ref, x_ref, wpt, bp, wnt, bn, wst, bs, y_ref, xs_ref):
    cc = c_ref[0, 0]
    sq = jnp.sqrt(cc)
    x = x_ref[...]
    nrm = jnp.sqrt(jnp.sum(x * x, axis=-1, keepdims=True))
    x = x / (nrm + 1e-08)
    h = jnp.dot(_log0(x, sq), wpt[...], preferred_element_type=jnp.float32) + bp[...]
    x0 = _exp0(h, sq)
    t = _log0(x0, sq)
    y_ref[...], xs_ref[...] = _two_branch(t, wnt[...], bn[...], wst[...], bs[...], sq)


def _layer_update(c_ref, xs_ref, y_ref, p_ref, d_ref, nc):
    cc = c_ref[0, 0]
    sq = jnp.sqrt(cc)
    y = y_ref[...]
    s = y
    for k in range(nc):
        s = s + p_ref[k]
    deg = d_ref[0, :, 0:1]
    for k in range(1, nc):
        deg = deg + d_ref[k, :, 0:1]
    deg = jnp.maximum(deg + 1.0, 1.0)
    agg = s / deg
    xn = _mobius_add(xs_ref[...], agg, cc)
    xt = jnp.tanh(_log0(xn, sq))
    return _exp0(xt, sq), sq


def _mid_body(c_ref, xs_ref, y_ref, p_ref, d_ref, wnt, bn, wst, bs, y1_ref, xs1_ref, *, nc):
    x1, sq = _layer_update(c_ref, xs_ref, y_ref, p_ref, d_ref, nc)
    t = _log0(x1, sq)
    y1_ref[...], xs1_ref[...] = _two_branch(t, wnt[...], bn[...], wst[...], bs[...], sq)


def _post_body(c_ref, xs_ref, y_ref, p_ref, d_ref, w1t, b1, w2t, b2, out_ref, *, nc):
    x2, sq = _layer_update(c_ref, xs_ref, y_ref, p_ref, d_ref, nc)
    t = _log0(x2, sq)
    h = jnp.tanh(jnp.dot(t, w1t[...], preferred_element_type=jnp.float32) + b1[...])
    out_ref[...] = jnp.dot(h, w2t[...], preferred_element_type=jnp.float32) + b2[...]


def _row_spec(blk):
    return pl.BlockSpec((blk, LANE), lambda i: (i, 0))


def _w_spec():
    return pl.BlockSpec((LANE, LANE), lambda i: (0, 0))


def _b_spec():
    return pl.BlockSpec((LANE,), lambda i: (0,))


def _smem_spec():
    return pl.BlockSpec(memory_space=pltpu.SMEM)


def _p_spec(nc, blk):
    return pl.BlockSpec((nc, blk, LANE), lambda i: (0, i, 0))


def _d_spec(nc, blk):
    return pl.BlockSpec((nc, blk, LANE), lambda i: (0, i, 0))


def _make_seg_sum(npad, nch0, nch1, nc, ns):
    rows_pt = npad // ns
    mesh = plsc.VectorSubcoreMesh(core_axis_name="c", subcore_axis_name="s")

    out_type = [jax.ShapeDtypeStruct((nc, npad, LANE), jnp.float32)]
    scratch = [
        pltpu.VMEM_SHARED((npad, LANE), jnp.float32),
        pltpu.VMEM((LANE,), jnp.int32),
        pltpu.VMEM((LANE,), jnp.int32),
        pltpu.VMEM((LANE,), jnp.int32),
        pltpu.VMEM((LANE,), jnp.int32),
        pltpu.VMEM((LANE, LANE), jnp.float32),
        pltpu.VMEM((LANE, LANE), jnp.float32),
        pltpu.SemaphoreType.DMA,
        pltpu.SemaphoreType.DMA,
        pltpu.SemaphoreType.DMA,
        pltpu.SemaphoreType.DMA,
    ]

    @functools.partial(pl.kernel, out_type=out_type, mesh=mesh,
                       scratch_types=scratch)
    def seg(y_hbm, src0_hbm, dst0_hbm, src1_hbm, dst1_hbm, z_hbm, agg_out,
            agg_sh, si0, si1, di0, di1, r0, r1, sem0, sem1, isem0, isem1):
        cid = lax.axis_index("c")
        sid = lax.axis_index("s")
        base = sid * rows_pt

        pltpu.sync_copy(z_hbm, agg_sh.at[pl.ds(base, rows_pt)])
        plsc.subcore_barrier()

        si = (si0, si1)
        di = (di0, di1)
        rows = (r0, r1)
        sems = (sem0, sem1)
        isems = (isem0, isem1)

        def edge_loop(src_hbm, dst_hbm, n_chunks):
            pltpu.sync_copy(src_hbm.at[sid, 0], si0)
            pltpu.sync_copy(dst_hbm.at[sid, 0], di0)
            pltpu.async_copy(y_hbm.at[si0], r0, sem0)
            if n_chunks > 1:
                pltpu.async_copy(src_hbm.at[sid, 1], si1, isem1)
                pltpu.async_copy(dst_hbm.at[sid, 1], di1, isem1)

            def loop_body(i2, carry):
                for b in (0, 1):
                    j = i2 * 2 + b
                    nb = 1 - b

                    pltpu.make_async_copy(y_hbm.at[si[b]], rows[b],
                                          sems[b]).wait()

                    @pl.when(j + 1 < n_chunks)
                    def _():
                        pltpu.make_async_copy(src_hbm.at[sid, j + 1], si[nb],
                                              isems[nb]).wait()
                        pltpu.make_async_copy(dst_hbm.at[sid, j + 1], di[nb],
                                              isems[nb]).wait()
                        pltpu.async_copy(y_hbm.at[si[nb]], rows[nb], sems[nb])

                    pltpu.sync_copy(rows[b], agg_sh.at[di[b]], add=True)

                    @pl.when(j + 2 < n_chunks)
                    def _():
                        pltpu.async_copy(src_hbm.at[sid, j + 2], si[b],
                                         isems[b])
                        pltpu.async_copy(dst_hbm.at[sid, j + 2], di[b],
                                         isems[b])
                return carry

            lax.fori_loop(0, n_chunks // 2, loop_body, 0)

        @pl.when(cid == 0)
        def _():
            edge_loop(src0_hbm, dst0_hbm, nch0)

        @pl.when(cid == 1)
        def _():
            edge_loop(src1_hbm, dst1_hbm, nch1)

        plsc.subcore_barrier()

        pltpu.sync_copy(agg_sh.at[pl.ds(base, rows_pt)],
                        agg_out.at[cid, pl.ds(base, rows_pt)])

    return seg


def _make_deg(npad, n_chunks, nc, ns):
    rows_pt = npad // ns
    mesh = plsc.VectorSubcoreMesh(core_axis_name="c", subcore_axis_name="s")

    @functools.partial(
        pl.kernel,
        out_type=[jax.ShapeDtypeStruct((nc, npad, LANE), jnp.float32)],
        mesh=mesh,
        scratch_types=[
            pltpu.VMEM_SHARED((npad, LANE), jnp.float32),
            pltpu.VMEM((LANE,), jnp.int32),
            pltpu.VMEM((LANE,), jnp.int32),
            pltpu.VMEM((LANE, LANE), jnp.float32),
            pltpu.SemaphoreType.DMA,
            pltpu.SemaphoreType.DMA,
        ])
    def deg(dst_hbm, z_hbm, ones_hbm, deg_out,
            deg_sh, di0, di1, ones_v, sem0, sem1):
        cid = lax.axis_index("c")
        sid = lax.axis_index("s")
        wid = sid * nc + cid
        base = sid * rows_pt

        pltpu.sync_copy(z_hbm, deg_sh.at[pl.ds(base, rows_pt)])
        pltpu.sync_copy(ones_hbm, ones_v)
        plsc.subcore_barrier()

        di = (di0, di1)
        sems = (sem0, sem1)
        pltpu.async_copy(dst_hbm.at[wid, 0], di0, sem0)

        def loop_body(i2, carry):
            for b in (0, 1):
                j = i2 * 2 + b
                nb = 1 - b

                @pl.when(j + 1 < n_chunks)
                def _():
                    pltpu.async_copy(dst_hbm.at[wid, j + 1], di[nb], sems[nb])

                pltpu.make_async_copy(dst_hbm.at[wid, j], di[b], sems[b]).wait()
                pltpu.sync_copy(ones_v, deg_sh.at[di[b]], add=True)
            return carry

        lax.fori_loop(0, n_chunks // 2, loop_body, 0)
        plsc.subcore_barrier()
        pltpu.sync_copy(deg_sh.at[pl.ds(base, rows_pt)],
                        deg_out.at[cid, pl.ds(base, rows_pt)])

    return deg


def kernel(x, c, Wp, bp, Ws0, bs0, Wn0, bn0, Ws1, bs1, Wn1, bn1, W1, b1,
           W2, b2, edge_index):
    N, D = x.shape
    H = Wp.shape[0]
    Cout = W2.shape[0]
    E = edge_index.shape[1]

    info = plsc.get_sparse_core_info()
    nc, ns = info.num_cores, info.num_subcores
    nw = nc * ns

    rows_pt = -(-(N + 1) // (ns * 8)) * 8
    npad = ns * rows_pt
    idt = edge_index.dtype

    n_chunks = max(2, -(-E // (nw * 2 * LANE)) * 2)
    pad_e = nw * n_chunks * LANE - E
    dstp = jnp.concatenate(
        [edge_index[1], jnp.full((pad_e,), N, idt)]).astype(jnp.int32).reshape(
            nw, n_chunks, LANE)

    frac0 = 0.75
    ch_tot = -(-E // LANE)
    nch0 = max(2, -(-int(ch_tot * frac0) // (ns * 2)) * 2)
    cap0 = ns * nch0 * LANE
    rem = max(0, E - cap0)
    nch1 = max(2, -(-rem // (LANE * ns * 2)) * 2)
    cap1 = ns * nch1 * LANE
    pad_s = cap0 + cap1 - E
    src_all = jnp.concatenate(
        [edge_index[0], jnp.full((pad_s,), N, idt)]).astype(jnp.int32)
    dst_all = jnp.concatenate(
        [edge_index[1], jnp.full((pad_s,), N, idt)]).astype(jnp.int32)
    src0 = src_all[:cap0].reshape(ns, nch0, LANE)
    dst0 = dst_all[:cap0].reshape(ns, nch0, LANE)
    src1 = src_all[cap0:].reshape(ns, nch1, LANE)
    dst1 = dst_all[cap0:].reshape(ns, nch1, LANE)

    xp = jnp.zeros((npad, D), jnp.float32).at[:N].set(x)
    c2 = jnp.reshape(c, (1, 1)).astype(jnp.float32)
    z128 = jnp.zeros((rows_pt, LANE), jnp.float32)
    o128 = jnp.ones((LANE, LANE), jnp.float32)

    w2pad = jnp.zeros((LANE, H), jnp.float32).at[:Cout].set(W2)
    b2pad = jnp.zeros((LANE,), jnp.float32).at[:Cout].set(b2)

    blk = 2 * rows_pt
    grid = (npad // blk,)

    f32 = jnp.float32
    rows2 = jax.ShapeDtypeStruct((npad, LANE), f32)

    y0, xs0 = pl.pallas_call(
        _pre_body,
        grid=grid,
        in_specs=[_smem_spec(), _row_spec(blk), _w_spec(), _b_spec(),
                  _w_spec(), _b_spec(), _w_spec(), _b_spec()],
        out_specs=[_row_spec(blk), _row_spec(blk)],
        out_shape=[rows2, rows2],
    )(c2, xp, Wp.T, bp, Wn0.T, bn0, Ws0.T, bs0)

    degk = _make_deg(npad, n_chunks, nc, ns)
    d0 = degk(dstp, z128, o128)
    if isinstance(d0, (list, tuple)):
        d0 = d0[0]

    seg0 = _make_seg_sum(npad, nch0, nch1, nc, ns)
    p0 = seg0(y0, src0, dst0, src1, dst1, z128)
    if isinstance(p0, (list, tuple)):
        p0 = p0[0]

    y1, xs1 = pl.pallas_call(
        functools.partial(_mid_body, nc=nc),
        grid=grid,
        in_specs=[_smem_spec(), _row_spec(blk), _row_spec(blk),
                  _p_spec(nc, blk), _d_spec(nc, blk),
                  _w_spec(), _b_spec(), _w_spec(), _b_spec()],
        out_specs=[_row_spec(blk), _row_spec(blk)],
        out_shape=[rows2, rows2],
    )(c2, xs0, y0, p0, d0, Wn1.T, bn1, Ws1.T, bs1)

    seg1 = _make_seg_sum(npad, nch0, nch1, nc, ns)
    p1 = seg1(y1, src0, dst0, src1, dst1, z128)
    if isinstance(p1, (list, tuple)):
        p1 = p1[0]

    out = pl.pallas_call(
        functools.partial(_post_body, nc=nc),
        grid=grid,
        in_specs=[_smem_spec(), _row_spec(blk), _row_spec(blk),
                  _p_spec(nc, blk), _d_spec(nc, blk),
                  _w_spec(), _b_spec(), _w_spec(), _b_spec()],
        out_specs=[_row_spec(blk)],
        out_shape=[rows2],
    )(c2, xs1, y1, p1, d0, W1.T, b1, w2pad.T, b2pad)
    if isinstance(out, (list, tuple)):
        out = out[0]

    return out[:N, :Cout]


# speedup vs baseline: 1.5381x; 1.0842x over previous
import functools

import jax
import jax.numpy as jnp
from jax import lax
from jax.experimental import pallas as pl
from jax.experimental.pallas import tpu as pltpu
from jax.experimental.pallas import tpu_sc as plsc

EPS = 1e-05
LANE = 128


def _log0(y, sqrt_c):
    diff = y / (1.0 + EPS)
    dn = jnp.sqrt(jnp.sum(diff * diff, axis=-1, keepdims=True))
    dn = jnp.maximum(dn, EPS)
    arg = jnp.minimum(sqrt_c * dn, 1.0 - 1e-06)
    atanh = 0.5 * jnp.log((1.0 + arg) / (1.0 - arg))
    return (atanh / (sqrt_c * dn)) * diff


def _exp0(v, sqrt_c):
    vn = jnp.sqrt(jnp.sum(v * v, axis=-1, keepdims=True))
    vn = jnp.maximum(vn, EPS)
    factor = jnp.tanh(sqrt_c * vn) / (sqrt_c * vn)
    return (factor * v) / (1.0 + EPS)


def _mobius_add(x, y, c):
    x2 = jnp.sum(x * x, axis=-1, keepdims=True)
    y2 = jnp.sum(y * y, axis=-1, keepdims=True)
    xy = jnp.sum(x * y, axis=-1, keepdims=True)
    num = (1.0 + 2.0 * c * xy + c * y2) * x + (1.0 - c * x2) * y
    den = 1.0 + 2.0 * c * xy + c * x2 * y2
    return num / (den + EPS)


def _two_branch(t, wnt, bn, wst, bs, sqrt_c):
    y = _exp0(jnp.dot(t, wnt, preferred_element_type=jnp.float32) + bn, sqrt_c)
    xs = _exp0(jnp.dot(t, wst, preferred_element_type=jnp.float32) + bs, sqrt_c)
    return y, xs


def _pre_body(c_ref, x_ref, wpt, bp, wnt, bn, wst, bs, y_ref, xs_ref):
    cc = c_ref[0, 0]
    sq = jnp.sqrt(cc)
    x = x_ref[...]
    nrm = jnp.sqrt(jnp.sum(x * x, axis=-1, keepdims=True))
    x = x / (nrm + 1e-08)
    h = jnp.dot(_log0(x, sq), wpt[...], preferred_element_type=jnp.float32) + bp[...]
    x0 = _exp0(h, sq)
    t = _log0(x0, sq)
    y_ref[...], xs_ref[...] = _two_branch(t, wnt[...], bn[...], wst[...], bs[...], sq)


def _layer_update(c_ref, xs_ref, y_ref, p_ref, d_ref, nc):
    cc = c_ref[0, 0]
    sq = jnp.sqrt(cc)
    y = y_ref[...]
    s = y
    for k in range(nc):
        s = s + p_ref[k]
    deg = d_ref[0, :, 0:1]
    for k in range(1, nc):
        deg = deg + d_ref[k, :, 0:1]
    deg = jnp.maximum(deg + 1.0, 1.0)
    agg = s / deg
    xn = _mobius_add(xs_ref[...], agg, cc)
    xt = jnp.tanh(_log0(xn, sq))
    return _exp0(xt, sq), sq


def _mid_body(c_ref, xs_ref, y_ref, p_ref, d_ref, wnt, bn, wst, bs, y1_ref, xs1_ref, *, nc):
    x1, sq = _layer_update(c_ref, xs_ref, y_ref, p_ref, d_ref, nc)
    t = _log0(x1, sq)
    y1_ref[...], xs1_ref[...] = _two_branch(t, wnt[...], bn[...], wst[...], bs[...], sq)


def _post_body(c_ref, xs_ref, y_ref, p_ref, d_ref, w1t, b1, w2t, b2, out_ref, *, nc):
    x2, sq = _layer_update(c_ref, xs_ref, y_ref, p_ref, d_ref, nc)
    t = _log0(x2, sq)
    h = jnp.tanh(jnp.dot(t, w1t[...], preferred_element_type=jnp.float32) + b1[...])
    out_ref[...] = jnp.dot(h, w2t[...], preferred_element_type=jnp.float32) + b2[...]


def _row_spec(blk):
    return pl.BlockSpec((blk, LANE), lambda i: (i, 0))


def _w_spec():
    return pl.BlockSpec((LANE, LANE), lambda i: (0, 0))


def _b_spec():
    return pl.BlockSpec((LANE,), lambda i: (0,))


def _smem_spec():
    return pl.BlockSpec(memory_space=pltpu.SMEM)


def _p_spec(nc, blk):
    return pl.BlockSpec((nc, blk, LANE), lambda i: (0, i, 0))


def _d_spec(nc, blk):
    return pl.BlockSpec((nc, blk, DEGW), lambda i: (0, i, 0))


def _make_seg_sum(npad, nch0, nch1, nc, ns):
    rows_pt = npad // ns
    mesh = plsc.VectorSubcoreMesh(core_axis_name="c", subcore_axis_name="s")

    out_type = [jax.ShapeDtypeStruct((nc, npad, LANE), jnp.float32)]
    scratch = [
        pltpu.VMEM_SHARED((npad, LANE), jnp.float32),
        pltpu.VMEM((LANE,), jnp.int32),
        pltpu.VMEM((LANE,), jnp.int32),
        pltpu.VMEM((LANE,), jnp.int32),
        pltpu.VMEM((LANE,), jnp.int32),
        pltpu.VMEM((LANE, LANE), jnp.float32),
        pltpu.VMEM((LANE, LANE), jnp.float32),
        pltpu.SemaphoreType.DMA,
        pltpu.SemaphoreType.DMA,
    ]

    @functools.partial(pl.kernel, out_type=out_type, mesh=mesh,
                       scratch_types=scratch)
    def seg(y_hbm, src0_hbm, dst0_hbm, src1_hbm, dst1_hbm, z_hbm, agg_out,
            agg_sh, si0, si1, di0, di1, r0, r1, sem0, sem1):
        cid = lax.axis_index("c")
        sid = lax.axis_index("s")
        base = sid * rows_pt

        pltpu.sync_copy(z_hbm, agg_sh.at[pl.ds(base, rows_pt)])
        plsc.subcore_barrier()

        si = (si0, si1)
        di = (di0, di1)
        rows = (r0, r1)
        sems = (sem0, sem1)

        def edge_loop(src_hbm, dst_hbm, n_chunks):
            pltpu.sync_copy(src_hbm.at[sid, 0], si0)
            pltpu.sync_copy(dst_hbm.at[sid, 0], di0)
            pltpu.async_copy(y_hbm.at[si0], r0, sem0)

            def loop_body(i2, carry):
                for b in (0, 1):
                    j = i2 * 2 + b
                    nb = 1 - b

                    @pl.when(j + 1 < n_chunks)
                    def _():
                        pltpu.sync_copy(src_hbm.at[sid, j + 1], si[nb])
                        pltpu.sync_copy(dst_hbm.at[sid, j + 1], di[nb])
                        pltpu.async_copy(y_hbm.at[si[nb]], rows[nb], sems[nb])

                    pltpu.make_async_copy(y_hbm.at[si[b]], rows[b],
                                          sems[b]).wait()
                    pltpu.sync_copy(rows[b], agg_sh.at[di[b]], add=True)
                return carry

            lax.fori_loop(0, n_chunks // 2, loop_body, 0)

        @pl.when(cid == 0)
        def _():
            edge_loop(src0_hbm, dst0_hbm, nch0)

        @pl.when(cid == 1)
        def _():
            edge_loop(src1_hbm, dst1_hbm, nch1)

        plsc.subcore_barrier()

        pltpu.sync_copy(agg_sh.at[pl.ds(base, rows_pt)],
                        agg_out.at[cid, pl.ds(base, rows_pt)])

    return seg


DEGW = LANE


def _make_deg(npad, n_chunks, nc, ns):
    rows_pt = npad // ns
    mesh = plsc.VectorSubcoreMesh(core_axis_name="c", subcore_axis_name="s")

    @functools.partial(
        pl.kernel,
        out_type=[jax.ShapeDtypeStruct((nc, npad, DEGW), jnp.float32)],
        mesh=mesh,
        scratch_types=[
            pltpu.VMEM_SHARED((npad, DEGW), jnp.float32),
            pltpu.VMEM((LANE,), jnp.int32),
            pltpu.VMEM((LANE,), jnp.int32),
            pltpu.VMEM((LANE, DEGW), jnp.float32),
            pltpu.SemaphoreType.DMA,
            pltpu.SemaphoreType.DMA,
        ])
    def deg(dst_hbm, z_hbm, ones_hbm, deg_out,
            deg_sh, di0, di1, ones_v, sem0, sem1):
        cid = lax.axis_index("c")
        sid = lax.axis_index("s")
        wid = sid * nc + cid
        base = sid * rows_pt

        pltpu.sync_copy(z_hbm, deg_sh.at[pl.ds(base, rows_pt)])
        pltpu.sync_copy(ones_hbm, ones_v)
        plsc.subcore_barrier()

        di = (di0, di1)
        sems = (sem0, sem1)
        pltpu.async_copy(dst_hbm.at[wid, 0], di0, sem0)

        def loop_body(i2, carry):
            for b in (0, 1):
                j = i2 * 2 + b
                nb = 1 - b

                @pl.when(j + 1 < n_chunks)
                def _():
                    pltpu.async_copy(dst_hbm.at[wid, j + 1], di[nb], sems[nb])

                pltpu.make_async_copy(dst_hbm.at[wid, j], di[b], sems[b]).wait()
                pltpu.sync_copy(ones_v, deg_sh.at[di[b]], add=True)
            return carry

        lax.fori_loop(0, n_chunks // 2, loop_body, 0)
        plsc.subcore_barrier()
        pltpu.sync_copy(deg_sh.at[pl.ds(base, rows_pt)],
                        deg_out.at[cid, pl.ds(base, rows_pt)])

    return deg


def kernel(x, c, Wp, bp, Ws0, bs0, Wn0, bn0, Ws1, bs1, Wn1, bn1, W1, b1,
           W2, b2, edge_index):
    N, D = x.shape
    H = Wp.shape[0]
    Cout = W2.shape[0]
    E = edge_index.shape[1]

    info = plsc.get_sparse_core_info()
    nc, ns = info.num_cores, info.num_subcores
    nw = nc * ns

    rows_pt = -(-(N + 1) // (ns * 8)) * 8
    npad = ns * rows_pt
    idt = edge_index.dtype

    n_chunks = max(2, -(-E // (nw * 2 * LANE)) * 2)
    pad_e = nw * n_chunks * LANE - E
    dstp = jnp.concatenate(
        [edge_index[1], jnp.full((pad_e,), N, idt)]).astype(jnp.int32).reshape(
            nw, n_chunks, LANE)

    frac0 = 0.75
    ch_tot = -(-E // LANE)
    nch0 = max(2, -(-int(ch_tot * frac0) // (ns * 2)) * 2)
    cap0 = ns * nch0 * LANE
    rem = max(0, E - cap0)
    nch1 = max(2, -(-rem // (LANE * ns * 2)) * 2)
    cap1 = ns * nch1 * LANE
    pad_s = cap0 + cap1 - E
    src_all = jnp.concatenate(
        [edge_index[0], jnp.full((pad_s,), N, idt)]).astype(jnp.int32)
    dst_all = jnp.concatenate(
        [edge_index[1], jnp.full((pad_s,), N, idt)]).astype(jnp.int32)
    src0 = src_all[:cap0].reshape(ns, nch0, LANE)
    dst0 = dst_all[:cap0].reshape(ns, nch0, LANE)
    src1 = src_all[cap0:].reshape(ns, nch1, LANE)
    dst1 = dst_all[cap0:].reshape(ns, nch1, LANE)

    xp = jnp.zeros((npad, D), jnp.float32).at[:N].set(x)
    c2 = jnp.reshape(c, (1, 1)).astype(jnp.float32)
    z64 = jnp.zeros((rows_pt, DEGW), jnp.float32)
    o64 = jnp.ones((LANE, DEGW), jnp.float32)

    w2pad = jnp.zeros((LANE, H), jnp.float32).at[:Cout].set(W2)
    b2pad = jnp.zeros((LANE,), jnp.float32).at[:Cout].set(b2)

    blk = 2 * rows_pt
    grid = (npad // blk,)

    f32 = jnp.float32
    rows2 = jax.ShapeDtypeStruct((npad, LANE), f32)

    y0, xs0 = pl.pallas_call(
        _pre_body,
        grid=grid,
        in_specs=[_smem_spec(), _row_spec(blk), _w_spec(), _b_spec(),
                  _w_spec(), _b_spec(), _w_spec(), _b_spec()],
        out_specs=[_row_spec(blk), _row_spec(blk)],
        out_shape=[rows2, rows2],
    )(c2, xp, Wp.T, bp, Wn0.T, bn0, Ws0.T, bs0)

    degk = _make_deg(npad, n_chunks, nc, ns)
    d0 = degk(dstp, z64, o64)
    if isinstance(d0, (list, tuple)):
        d0 = d0[0]

    z128 = jnp.broadcast_to(jnp.minimum(d0[0, 0, 0], 0.0),
                            (rows_pt, LANE))

    seg0 = _make_seg_sum(npad, nch0, nch1, nc, ns)
    p0 = seg0(y0, src0, dst0, src1, dst1, z128)
    if isinstance(p0, (list, tuple)):
        p0 = p0[0]

    y1, xs1 = pl.pallas_call(
        functools.partial(_mid_body, nc=nc),
        grid=grid,
        in_specs=[_smem_spec(), _row_spec(blk), _row_spec(blk),
                  _p_spec(nc, blk), _d_spec(nc, blk),
                  _w_spec(), _b_spec(), _w_spec(), _b_spec()],
        out_specs=[_row_spec(blk), _row_spec(blk)],
        out_shape=[rows2, rows2],
    )(c2, xs0, y0, p0, d0, Wn1.T, bn1, Ws1.T, bs1)

    seg1 = _make_seg_sum(npad, nch0, nch1, nc, ns)
    p1 = seg1(y1, src0, dst0, src1, dst1, z128)
    if isinstance(p1, (list, tuple)):
        p1 = p1[0]

    out = pl.pallas_call(
        functools.partial(_post_body, nc=nc),
        grid=grid,
        in_specs=[_smem_spec(), _row_spec(blk), _row_spec(blk),
                  _p_spec(nc, blk), _d_spec(nc, blk),
                  _w_spec(), _b_spec(), _w_spec(), _b_spec()],
        out_specs=[_row_spec(blk)],
        out_shape=[rows2],
    )(c2, xs1, y1, p1, d0, W1.T, b1, w2pad.T, b2pad)
    if isinstance(out, (list, tuple)):
        out = out[0]

    return out[:N, :Cout]
